# Initial kernel scaffold; baseline (speedup 1.0000x reference)
import jax
import jax.numpy as jnp
from jax.experimental import pallas as pl


def kernel(x, edge_index, batch, W1, att_src1, att_dst1, b1, W2, att_src2, att_dst2, b2):
    raise NotImplementedError("write your pallas kernel here")


# speedup vs baseline: 1.1975x; 1.1975x over previous
import functools

import jax
import jax.numpy as jnp
from jax.experimental import pallas as pl
from jax.experimental.pallas import tpu as pltpu

N = 50000
E = 800000
G = 1000
IN_DIM = 9
HIDDEN = 64
HEADS = 4


def _att_proj(att_src, att_dst):
    heads = att_src.shape[0]
    hh = heads * HIDDEN
    rows = jnp.arange(hh, dtype=jnp.int32)
    head_of_row = rows // HIDDEN
    A = jnp.zeros((hh, 2 * heads), dtype=jnp.float32)
    A = A.at[rows, head_of_row].set(att_src.reshape(-1))
    A = A.at[rows, heads + head_of_row].set(att_dst.reshape(-1))
    return A


def _dense1_body(x_ref, w_ref, a_ref, h_ref, asd_ref):
    h = jnp.dot(x_ref[...], w_ref[...], preferred_element_type=jnp.float32)
    h_ref[...] = h
    asd_ref[...] = jnp.dot(h, a_ref[...], preferred_element_type=jnp.float32)


def _dense1(x, W1, A1):
    R = 2000
    grid = (N // R,)
    hh = HEADS * HIDDEN
    return pl.pallas_call(
        _dense1_body,
        grid=grid,
        in_specs=[
            pl.BlockSpec((R, IN_DIM), lambda i: (i, 0)),
            pl.BlockSpec((IN_DIM, hh), lambda i: (0, 0)),
            pl.BlockSpec((hh, 2 * HEADS), lambda i: (0, 0)),
        ],
        out_specs=[
            pl.BlockSpec((R, hh), lambda i: (i, 0)),
            pl.BlockSpec((R, 2 * HEADS), lambda i: (i, 0)),
        ],
        out_shape=[
            jax.ShapeDtypeStruct((N, hh), jnp.float32),
            jax.ShapeDtypeStruct((N, 2 * HEADS), jnp.float32),
        ],
    )(x, W1, A1)


def _dense2_body(agg_ref, b_ref, w_ref, a_ref, h_ref, asd_ref):
    act = agg_ref[...] + b_ref[...]
    act = jnp.where(act > 0, act, jnp.exp(jnp.minimum(act, 0.0)) - 1.0)
    h = jnp.dot(act, w_ref[...], preferred_element_type=jnp.float32)
    h_ref[...] = h
    asd_ref[...] = jnp.dot(h, a_ref[...], preferred_element_type=jnp.float32)


def _dense2(agg1, b1, W2, A2):
    R = 2000
    grid = (N // R,)
    hh = HEADS * HIDDEN
    return pl.pallas_call(
        _dense2_body,
        grid=grid,
        in_specs=[
            pl.BlockSpec((R, hh), lambda i: (i, 0)),
            pl.BlockSpec((1, hh), lambda i: (0, 0)),
            pl.BlockSpec((hh, HIDDEN), lambda i: (0, 0)),
            pl.BlockSpec((HIDDEN, 2), lambda i: (0, 0)),
        ],
        out_specs=[
            pl.BlockSpec((R, HIDDEN), lambda i: (i, 0)),
            pl.BlockSpec((R, 2), lambda i: (i, 0)),
        ],
        out_shape=[
            jax.ShapeDtypeStruct((N, HIDDEN), jnp.float32),
            jax.ShapeDtypeStruct((N, 2), jnp.float32),
        ],
    )(agg1, b1.reshape(1, hh), W2, A2)


def _edge_phase_jnp(h, asd, src, dst, heads):
    n = h.shape[0]
    a_s = asd[:, :heads]
    a_d = asd[:, heads:]
    e = a_s[src] + a_d[dst]
    e = jnp.where(e > 0, e, 0.2 * e)
    w = jnp.exp(e)
    e_loop = a_s + a_d
    e_loop = jnp.where(e_loop > 0, e_loop, 0.2 * e_loop)
    w_loop = jnp.exp(e_loop)
    den = jax.ops.segment_sum(w, dst, num_segments=n) + w_loop
    hv = h.reshape(n, heads, HIDDEN)
    msg = hv[src] * (w / (den[dst] + 1e-16))[:, :, None]
    out = jax.ops.segment_sum(msg, dst, num_segments=n)
    out = out + hv * (w_loop / (den + 1e-16))[:, :, None]
    return out.reshape(n, heads * HIDDEN)


def kernel(x, edge_index, batch, W1, att_src1, att_dst1, b1, W2, att_src2, att_dst2, b2):
    src = edge_index[0]
    dst = edge_index[1]

    A1 = _att_proj(att_src1, att_dst1)
    h1, asd1 = _dense1(x, W1, A1)
    agg1 = _edge_phase_jnp(h1, asd1, src, dst, HEADS)

    A2 = _att_proj(att_src2, att_dst2)
    h2, asd2 = _dense2(agg1, b1, W2, A2)
    agg2 = _edge_phase_jnp(h2, asd2, src, dst, 1)

    act2 = agg2 + b2
    act2 = jnp.where(act2 > 0, act2, jnp.expm1(act2))
    s = jax.ops.segment_sum(act2, batch, num_segments=G)
    cnt = jax.ops.segment_sum(jnp.ones((N, 1), jnp.float32), batch, num_segments=G)
    return s / jnp.maximum(cnt, 1.0)


# speedup vs baseline: 1.4563x; 1.2161x over previous
import functools

import jax
import jax.numpy as jnp
from jax import lax
from jax.experimental import pallas as pl
from jax.experimental.pallas import tpu as pltpu
from jax.experimental.pallas import tpu_sc as plsc

N = 50000
E = 800000
G = 1000
IN_DIM = 9
HIDDEN = 64
HEADS = 4

NP = 50176
EP = 832000
K = 2000
NTILES = 32
EB = EP // NTILES
NB = EB // K


def _iota16():
    return lax.broadcasted_iota(jnp.int32, (16,), 0)


def _edge_w_kernel(H):
    mesh = plsc.VectorSubcoreMesh(core_axis_name="c", subcore_axis_name="s")
    STRIPE = NP * H // 16

    CH = 3136
    NCH = STRIPE // CH

    def body(src_hbm, dst_hbm, as_hbm, ad_hbm,
             w_hbm, den_hbm,
             srcb, dstb, gidx, didx, asg, adg, wb, zb, den_sh):
        c = lax.axis_index("c")
        s = lax.axis_index("s")

        def zrow(g, _):
            zb[pl.ds(g * 16, 16)] = jnp.zeros((16,), jnp.float32)
            return 0
        lax.fori_loop(0, CH // 16, zrow, 0, unroll=8)
        for q in range(NCH):
            pltpu.sync_copy(zb, den_sh.at[pl.ds(s * STRIPE + q * CH, CH)])
        plsc.subcore_barrier()

        tile = c * 16 + s

        def batch_body(j, _):
            base = tile * EB + j * K
            pltpu.sync_copy(src_hbm.at[pl.ds(base, K)], srcb)
            pltpu.sync_copy(dst_hbm.at[pl.ds(base, K)], dstb)

            for h in range(H):
                if H > 1:
                    def off(g, _):
                        sl = pl.ds(g * 16, 16)
                        gidx[sl] = srcb[sl] + h * NP
                        didx[sl] = dstb[sl] + h * NP
                        return 0
                    lax.fori_loop(0, K // 16, off, 0, unroll=8)
                    gi, di = gidx, didx
                else:
                    gi, di = srcb, dstb
                pltpu.sync_copy(as_hbm.at[gi], asg)
                pltpu.sync_copy(ad_hbm.at[di], adg)

                def compute(g, _):
                    sl = pl.ds(g * 16, 16)
                    t = asg[sl] + adg[sl]
                    wb[sl] = jnp.exp(jnp.maximum(t, 0.2 * t))
                    return 0
                lax.fori_loop(0, K // 16, compute, 0, unroll=8)

                pltpu.sync_copy(wb, w_hbm.at[pl.ds(h * EP + base, K)])
                pltpu.sync_copy(wb, den_sh.at[di], add=True)
            return 0

        lax.fori_loop(0, NB, batch_body, 0)
        plsc.subcore_barrier()
        for q in range(NCH):
            pltpu.sync_copy(den_sh.at[pl.ds(s * STRIPE + q * CH, CH)], zb)
            pltpu.sync_copy(zb, den_hbm.at[pl.ds(c * (NP * H) + s * STRIPE + q * CH, CH)])

    scratch = [
        pltpu.VMEM((K,), jnp.int32),
        pltpu.VMEM((K,), jnp.int32),
        pltpu.VMEM((K,), jnp.int32),
        pltpu.VMEM((K,), jnp.int32),
        pltpu.VMEM((K,), jnp.float32),
        pltpu.VMEM((K,), jnp.float32),
        pltpu.VMEM((K,), jnp.float32),
        pltpu.VMEM((CH,), jnp.float32),
        pltpu.VMEM_SHARED((NP * H,), jnp.float32),
    ]
    return pl.kernel(
        body,
        mesh=mesh,
        out_type=[
            jax.ShapeDtypeStruct((EP * H,), jnp.float32),
            jax.ShapeDtypeStruct((2 * NP * H,), jnp.float32),
        ],
        scratch_types=scratch,
    )


def _att_proj(att_src, att_dst):
    heads = att_src.shape[0]
    hh = heads * HIDDEN
    rows = jnp.arange(hh, dtype=jnp.int32)
    head_of_row = rows // HIDDEN
    A = jnp.zeros((hh, 2 * heads), dtype=jnp.float32)
    A = A.at[rows, head_of_row].set(att_src.reshape(-1))
    A = A.at[rows, heads + head_of_row].set(att_dst.reshape(-1))
    return A


def _dense1_body(x_ref, w_ref, a_ref, h_ref, as_ref, ad_ref, wl_ref):
    h = jnp.dot(x_ref[...], w_ref[...], preferred_element_type=jnp.float32)
    h_ref[...] = h
    asd = jnp.dot(h, a_ref[...], preferred_element_type=jnp.float32)
    a_s = asd[:, :HEADS]
    a_d = asd[:, HEADS:]
    as_ref[...] = a_s
    ad_ref[...] = a_d
    t = a_s + a_d
    wl_ref[...] = jnp.exp(jnp.maximum(t, 0.2 * t))


def _dense1(x_p, W1, A1):
    R = 1568
    grid = (NP // R,)
    hh = HEADS * HIDDEN
    return pl.pallas_call(
        _dense1_body,
        grid=grid,
        in_specs=[
            pl.BlockSpec((R, IN_DIM), lambda i: (i, 0)),
            pl.BlockSpec((IN_DIM, hh), lambda i: (0, 0)),
            pl.BlockSpec((hh, 2 * HEADS), lambda i: (0, 0)),
        ],
        out_specs=[
            pl.BlockSpec((R, hh), lambda i: (i, 0)),
            pl.BlockSpec((R, HEADS), lambda i: (i, 0)),
            pl.BlockSpec((R, HEADS), lambda i: (i, 0)),
            pl.BlockSpec((R, HEADS), lambda i: (i, 0)),
        ],
        out_shape=[
            jax.ShapeDtypeStruct((NP, hh), jnp.float32),
            jax.ShapeDtypeStruct((NP, HEADS), jnp.float32),
            jax.ShapeDtypeStruct((NP, HEADS), jnp.float32),
            jax.ShapeDtypeStruct((NP, HEADS), jnp.float32),
        ],
    )(x_p, W1, A1)


def _rep_mat(heads, out_ch):
    cols = jnp.arange(heads * out_ch, dtype=jnp.int32)
    return (cols[None, :] // out_ch == jnp.arange(heads, dtype=jnp.int32)[:, None]
            ).astype(jnp.float32)


def _dense2_body(un_ref, h1_ref, wl_ref, dena_ref, denb_ref, rep_ref,
                 b_ref, w_ref, a_ref, h2_ref, as_ref, ad_ref, wl2_ref):
    den = dena_ref[...] + denb_ref[...] + wl_ref[...]
    rden = 1.0 / (den + 1e-16)
    rden_rep = jnp.dot(rden, rep_ref[...], preferred_element_type=jnp.float32)
    wl_rep = jnp.dot(wl_ref[...], rep_ref[...], preferred_element_type=jnp.float32)
    act = (un_ref[...] + wl_rep * h1_ref[...]) * rden_rep + b_ref[...]
    act = jnp.where(act > 0, act, jnp.exp(jnp.minimum(act, 0.0)) - 1.0)
    h2 = jnp.dot(act, w_ref[...], preferred_element_type=jnp.float32)
    h2_ref[...] = h2
    asd = jnp.dot(h2, a_ref[...], preferred_element_type=jnp.float32)
    a_s = asd[:, :1]
    a_d = asd[:, 1:]
    as_ref[...] = a_s
    ad_ref[...] = a_d
    t = a_s + a_d
    wl2_ref[...] = jnp.exp(jnp.maximum(t, 0.2 * t))


def _dense2(unnorm1, h1_p, wl1, den1a, den1b, b1, W2, A2):
    R = 1568
    grid = (NP // R,)
    hh = HEADS * HIDDEN
    rep = _rep_mat(HEADS, HIDDEN)
    return pl.pallas_call(
        _dense2_body,
        grid=grid,
        in_specs=[
            pl.BlockSpec((R, hh), lambda i: (i, 0)),
            pl.BlockSpec((R, hh), lambda i: (i, 0)),
            pl.BlockSpec((R, HEADS), lambda i: (i, 0)),
            pl.BlockSpec((R, HEADS), lambda i: (i, 0)),
            pl.BlockSpec((R, HEADS), lambda i: (i, 0)),
            pl.BlockSpec((HEADS, hh), lambda i: (0, 0)),
            pl.BlockSpec((1, hh), lambda i: (0, 0)),
            pl.BlockSpec((hh, HIDDEN), lambda i: (0, 0)),
            pl.BlockSpec((HIDDEN, 2), lambda i: (0, 0)),
        ],
        out_specs=[
            pl.BlockSpec((R, HIDDEN), lambda i: (i, 0)),
            pl.BlockSpec((R, 1), lambda i: (i, 0)),
            pl.BlockSpec((R, 1), lambda i: (i, 0)),
            pl.BlockSpec((R, 1), lambda i: (i, 0)),
        ],
        out_shape=[
            jax.ShapeDtypeStruct((NP, HIDDEN), jnp.float32),
            jax.ShapeDtypeStruct((NP, 1), jnp.float32),
            jax.ShapeDtypeStruct((NP, 1), jnp.float32),
            jax.ShapeDtypeStruct((NP, 1), jnp.float32),
        ],
    )(unnorm1, h1_p, wl1, den1a, den1b, rep, b1.reshape(1, hh), W2, A2)


def kernel(x, edge_index, batch, W1, att_src1, att_dst1, b1, W2, att_src2, att_dst2, b2):
    src = edge_index[0]
    dst = edge_index[1]
    pad_src = jnp.full((EP - E,), N, jnp.int32)
    pad_dst = jnp.full((EP - E,), NP - 1, jnp.int32)
    src_p = jnp.concatenate([src, pad_src])
    dst_p = jnp.concatenate([dst, pad_dst])
    x_p = jnp.zeros((NP, IN_DIM), jnp.float32).at[:N].set(x)

    A1 = _att_proj(att_src1, att_dst1)
    h1_p, as1, ad1, wl1_p = _dense1(x_p, W1, A1)

    w1f, den1f = _edge_w_kernel(HEADS)(
        src_p, dst_p, as1.T.reshape(-1), ad1.T.reshape(-1))
    w1 = w1f.reshape(HEADS, EP).T
    den1 = den1f.reshape(2, HEADS, NP).transpose(0, 2, 1)

    h1v = h1_p[:N].reshape(N, HEADS, HIDDEN)
    msg = h1v[src] * w1[:E, :, None]
    un1 = jax.ops.segment_sum(msg, dst, num_segments=N).reshape(N, HEADS * HIDDEN)
    un1_p = jnp.zeros((NP, HEADS * HIDDEN), jnp.float32).at[:N].set(un1)

    A2 = _att_proj(att_src2, att_dst2)
    h2_p, as2, ad2, wl2_p = _dense2(un1_p, h1_p, wl1_p, den1[0], den1[1], b1, W2, A2)

    w2f, den2f = _edge_w_kernel(1)(
        src_p, dst_p, as2.reshape(-1), ad2.reshape(-1))
    w2 = w2f.reshape(EP, 1)
    den2 = den2f.reshape(2, NP, 1)

    h2 = h2_p[:N]
    msg2 = h2[src] * w2[:E]
    un2 = jax.ops.segment_sum(msg2, dst, num_segments=N)
    den2t = den2[0, :N] + den2[1, :N] + wl2_p[:N]
    act2 = (un2 + wl2_p[:N] * h2) / (den2t + 1e-16) + b2
    act2 = jnp.where(act2 > 0, act2, jnp.expm1(act2))

    s = jax.ops.segment_sum(act2, batch, num_segments=G)
    cnt = jax.ops.segment_sum(jnp.ones((N, 1), jnp.float32), batch, num_segments=G)
    return s / jnp.maximum(cnt, 1.0)


# speedup vs baseline: 12.9900x; 8.9197x over previous
import functools

import jax
import jax.numpy as jnp
from jax import lax
from jax.experimental import pallas as pl
from jax.experimental.pallas import tpu as pltpu
from jax.experimental.pallas import tpu_sc as plsc

N = 50000
E = 800000
G = 1000
IN_DIM = 9
HIDDEN = 64
HEADS = 4

NP = 50176
EP = 832000
K = 2000
NTILES = 32
EB = EP // NTILES
NB = EB // K
CB = 16
R = 1568


def _edge_w_kernel(H):
    mesh = plsc.VectorSubcoreMesh(core_axis_name="c", subcore_axis_name="s")
    STRIPE = NP * H // 16
    CH = 3136
    NCH = STRIPE // CH

    def body(src_hbm, dst_hbm, as_hbm, ad_hbm,
             w_hbm, den_hbm,
             srcb, dstb, gidx, didx, asg, adg, wb, zb, den_sh):
        c = lax.axis_index("c")
        s = lax.axis_index("s")

        def zrow(g, _):
            zb[pl.ds(g * 16, 16)] = jnp.zeros((16,), jnp.float32)
            return 0
        lax.fori_loop(0, CH // 16, zrow, 0, unroll=8)
        for q in range(NCH):
            pltpu.sync_copy(zb, den_sh.at[pl.ds(s * STRIPE + q * CH, CH)])
        plsc.subcore_barrier()

        tile = c * 16 + s

        def batch_body(j, _):
            base = tile * EB + j * K
            pltpu.sync_copy(src_hbm.at[pl.ds(base, K)], srcb)
            pltpu.sync_copy(dst_hbm.at[pl.ds(base, K)], dstb)

            for h in range(H):
                if H > 1:
                    def off(g, _):
                        sl = pl.ds(g * 16, 16)
                        gidx[sl] = srcb[sl] + h * NP
                        didx[sl] = dstb[sl] + h * NP
                        return 0
                    lax.fori_loop(0, K // 16, off, 0, unroll=8)
                    gi, di = gidx, didx
                else:
                    gi, di = srcb, dstb
                pltpu.sync_copy(as_hbm.at[gi], asg)
                pltpu.sync_copy(ad_hbm.at[di], adg)

                def compute(g, _):
                    sl = pl.ds(g * 16, 16)
                    t = asg[sl] + adg[sl]
                    wb[sl] = jnp.exp(jnp.maximum(t, 0.2 * t))
                    return 0
                lax.fori_loop(0, K // 16, compute, 0, unroll=8)

                pltpu.sync_copy(wb, w_hbm.at[pl.ds(h * EP + base, K)])
                pltpu.sync_copy(wb, den_sh.at[di], add=True)
            return 0

        lax.fori_loop(0, NB, batch_body, 0)
        plsc.subcore_barrier()
        for q in range(NCH):
            pltpu.sync_copy(den_sh.at[pl.ds(s * STRIPE + q * CH, CH)], zb)
            pltpu.sync_copy(
                zb, den_hbm.at[pl.ds(c * (NP * H) + s * STRIPE + q * CH, CH)])

    scratch = [
        pltpu.VMEM((K,), jnp.int32),
        pltpu.VMEM((K,), jnp.int32),
        pltpu.VMEM((K,), jnp.int32),
        pltpu.VMEM((K,), jnp.int32),
        pltpu.VMEM((K,), jnp.float32),
        pltpu.VMEM((K,), jnp.float32),
        pltpu.VMEM((K,), jnp.float32),
        pltpu.VMEM((CH,), jnp.float32),
        pltpu.VMEM_SHARED((NP * H,), jnp.float32),
    ]
    return pl.kernel(
        body,
        mesh=mesh,
        out_type=[
            jax.ShapeDtypeStruct((EP * H,), jnp.float32),
            jax.ShapeDtypeStruct((2 * NP * H,), jnp.float32),
        ],
        scratch_types=scratch,
    )


def _msg_kernel(H):
    NCB = H * HIDDEN // CB
    HH = H * HIDDEN
    mesh = plsc.VectorSubcoreMesh(core_axis_name="c", subcore_axis_name="s")
    SROW = NP // 16
    WR = 784

    def body(src_hbm, dst_hbm, w_hbm, *rest):
        hcbs = rest[:NCB]
        un_hbm = rest[NCB]
        srcb, dstb, wbh, rows, zb, acc_sh = rest[NCB + 1:]
        c = lax.axis_index("c")
        s = lax.axis_index("s")
        tile = c * 16 + s

        def zrow(i, _):
            zb[i, pl.ds(0, 16)] = jnp.zeros((16,), jnp.float32)
            return 0
        lax.fori_loop(0, WR, zrow, 0, unroll=8)

        for cg in range(NCB):
            hd = cg * CB // HIDDEN
            for q in range(SROW // WR):
                pltpu.sync_copy(zb, acc_sh.at[pl.ds(s * SROW + q * WR, WR)])
            plsc.subcore_barrier()

            def batch_body(j, _):
                base = tile * EB + j * K
                pltpu.sync_copy(src_hbm.at[pl.ds(base, K)], srcb)
                pltpu.sync_copy(dst_hbm.at[pl.ds(base, K)], dstb)
                pltpu.sync_copy(w_hbm.at[pl.ds(hd * EP + base, K)],
                                wbh.at[pl.ds(0, K)])
                pltpu.sync_copy(hcbs[cg].at[srcb], rows)

                def mul(i, _):
                    w = wbh[pl.ds(i, 16)][0]
                    rows[i, pl.ds(0, 16)] = rows[i, pl.ds(0, 16)] * w
                    return 0
                lax.fori_loop(0, K, mul, 0, unroll=4)

                pltpu.sync_copy(rows, acc_sh.at[dstb], add=True)
                return 0

            lax.fori_loop(0, NB, batch_body, 0)
            plsc.subcore_barrier()

            for q in range(SROW // WR):
                row0 = s * SROW + q * WR
                pltpu.sync_copy(acc_sh.at[pl.ds(row0, WR)], rows.at[pl.ds(0, WR)])
                pltpu.sync_copy(rows.at[pl.ds(0, WR)],
                                un_hbm.at[pl.ds(c * NP + row0, WR),
                                          pl.ds(cg * CB, CB)])
            plsc.subcore_barrier()

    scratch = [
        pltpu.VMEM((K,), jnp.int32),
        pltpu.VMEM((K,), jnp.int32),
        pltpu.VMEM((K + 16,), jnp.float32),
        pltpu.VMEM((K, CB), jnp.float32),
        pltpu.VMEM((WR, CB), jnp.float32),
        pltpu.VMEM_SHARED((NP, CB), jnp.float32),
    ]
    return pl.kernel(
        body,
        mesh=mesh,
        out_type=jax.ShapeDtypeStruct((2 * NP, HH), jnp.float32),
        scratch_types=scratch,
        compiler_params=pltpu.CompilerParams(use_tc_tiling_on_sc=False),
    )


def _att_proj(att_src, att_dst):
    heads = att_src.shape[0]
    hh = heads * HIDDEN
    rows = jnp.arange(hh, dtype=jnp.int32)
    head_of_row = rows // HIDDEN
    A = jnp.zeros((hh, 2 * heads), dtype=jnp.float32)
    A = A.at[rows, head_of_row].set(att_src.reshape(-1))
    A = A.at[rows, heads + head_of_row].set(att_dst.reshape(-1))
    return A


NCB1 = HEADS * HIDDEN // CB
NCB2 = HIDDEN // CB


def _dense1_body(x_ref, w1_ref, a_ref, hrow_ref, *outs):
    hcb_refs = outs[:NCB1]
    asd_ref = outs[NCB1]
    wl_ref = outs[NCB1 + 1]
    h = jnp.dot(x_ref[...], w1_ref[...], preferred_element_type=jnp.float32)
    hrow_ref[...] = h
    for cg in range(NCB1):
        hcb_refs[cg][...] = h[:, cg * CB:(cg + 1) * CB]
    asd = jnp.dot(h, a_ref[...], preferred_element_type=jnp.float32)
    asd_ref[...] = asd
    t = asd[:, :HEADS] + asd[:, HEADS:]
    wl_ref[...] = jnp.exp(jnp.maximum(t, 0.2 * t))


def _dense1(x_p, W1, A1):
    grid = (NP // R,)
    hh = HEADS * HIDDEN
    blk = lambda cols: pl.BlockSpec((R, cols), lambda i: (i, 0))
    return pl.pallas_call(
        _dense1_body,
        grid=grid,
        in_specs=[
            blk(IN_DIM),
            pl.BlockSpec((IN_DIM, hh), lambda i: (0, 0)),
            pl.BlockSpec((hh, 2 * HEADS), lambda i: (0, 0)),
        ],
        out_specs=[blk(hh)] + [blk(CB)] * NCB1 + [blk(2 * HEADS), blk(HEADS)],
        out_shape=[jax.ShapeDtypeStruct((NP, hh), jnp.float32)]
        + [jax.ShapeDtypeStruct((NP, CB), jnp.float32)] * NCB1
        + [jax.ShapeDtypeStruct((NP, 2 * HEADS), jnp.float32),
           jax.ShapeDtypeStruct((NP, HEADS), jnp.float32)],
    )(x_p, W1, A1)


def _rep_mat(heads, out_ch):
    cols = jnp.arange(heads * out_ch, dtype=jnp.int32)
    return (cols[None, :] // out_ch == jnp.arange(heads, dtype=jnp.int32)[:, None]
            ).astype(jnp.float32)


def _dense2_body(una_ref, unb_ref, h1_ref, wl_ref, dena_ref, denb_ref,
                 rep_ref, b_ref, w2_ref, a_ref, *outs):
    h2row_ref = outs[0]
    h2cb_refs = outs[1:1 + NCB2]
    asd2_ref = outs[1 + NCB2]
    wl2_ref = outs[2 + NCB2]

    den = dena_ref[...] + denb_ref[...] + wl_ref[...]
    rden = 1.0 / (den + 1e-16)
    rden_rep = jnp.dot(rden, rep_ref[...], preferred_element_type=jnp.float32)
    wl_rep = jnp.dot(wl_ref[...], rep_ref[...], preferred_element_type=jnp.float32)

    act = (una_ref[...] + unb_ref[...]
           + wl_rep * h1_ref[...]) * rden_rep + b_ref[...]
    act = jnp.where(act > 0, act, jnp.exp(jnp.minimum(act, 0.0)) - 1.0)
    h2 = jnp.dot(act, w2_ref[...], preferred_element_type=jnp.float32)

    h2row_ref[...] = h2
    for cg in range(NCB2):
        h2cb_refs[cg][...] = h2[:, cg * CB:(cg + 1) * CB]
    asd = jnp.dot(h2, a_ref[...], preferred_element_type=jnp.float32)
    asd2_ref[...] = asd
    t = asd[:, :1] + asd[:, 1:]
    wl2_ref[...] = jnp.exp(jnp.maximum(t, 0.2 * t))


def _dense2(un1a, un1b, h1row, wl1, den1a, den1b, b1, W2, A2):
    grid = (NP // R,)
    hh = HEADS * HIDDEN
    rep = _rep_mat(HEADS, HIDDEN)
    blk = lambda cols: pl.BlockSpec((R, cols), lambda i: (i, 0))
    in_specs = [blk(hh), blk(hh), blk(hh),
                blk(HEADS), blk(HEADS), blk(HEADS),
                pl.BlockSpec((HEADS, hh), lambda i: (0, 0)),
                pl.BlockSpec((1, hh), lambda i: (0, 0)),
                pl.BlockSpec((hh, HIDDEN), lambda i: (0, 0)),
                pl.BlockSpec((HIDDEN, 2), lambda i: (0, 0))]
    return pl.pallas_call(
        _dense2_body,
        grid=grid,
        in_specs=in_specs,
        out_specs=[blk(HIDDEN)] + [blk(CB)] * NCB2 + [blk(2), blk(1)],
        out_shape=[jax.ShapeDtypeStruct((NP, HIDDEN), jnp.float32)]
        + [jax.ShapeDtypeStruct((NP, CB), jnp.float32)] * NCB2
        + [jax.ShapeDtypeStruct((NP, 2), jnp.float32),
           jax.ShapeDtypeStruct((NP, 1), jnp.float32)],
    )(un1a, un1b, h1row, wl1, den1a, den1b, rep, b1.reshape(1, hh), W2, A2)


def kernel(x, edge_index, batch, W1, att_src1, att_dst1, b1, W2, att_src2, att_dst2, b2):
    src = edge_index[0]
    dst = edge_index[1]
    pad_src = jnp.full((EP - E,), N, jnp.int32)
    pad_dst = jnp.full((EP - E,), NP - 1, jnp.int32)
    src_p = jnp.concatenate([src, pad_src])
    dst_p = jnp.concatenate([dst, pad_dst])
    x_p = jnp.zeros((NP, IN_DIM), jnp.float32).at[:N].set(x)

    A1 = _att_proj(att_src1, att_dst1)
    d1out = _dense1(x_p, W1, A1)
    h1row = d1out[0]
    hcb1 = d1out[1:1 + NCB1]
    asd1, wl1_p = d1out[1 + NCB1:]
    as1 = asd1[:, :HEADS]
    ad1 = asd1[:, HEADS:]

    w1f, den1f = _edge_w_kernel(HEADS)(
        src_p, dst_p, as1.T.reshape(-1), ad1.T.reshape(-1))
    den1 = den1f.reshape(2, HEADS, NP).transpose(0, 2, 1)

    un1f = _msg_kernel(HEADS)(src_p, dst_p, w1f, *hcb1)

    A2 = _att_proj(att_src2, att_dst2)
    d2out = _dense2(un1f[:NP], un1f[NP:], h1row, wl1_p, den1[0], den1[1],
                    b1, W2, A2)
    h2row = d2out[0]
    hcb2 = d2out[1:1 + NCB2]
    asd2, wl2_p = d2out[1 + NCB2:]
    as2 = asd2[:, :1]
    ad2 = asd2[:, 1:]

    w2f, den2f = _edge_w_kernel(1)(
        src_p, dst_p, as2.reshape(-1), ad2.reshape(-1))
    den2 = den2f.reshape(2, NP, 1)

    un2f = _msg_kernel(1)(src_p, dst_p, w2f, *hcb2)
    un2 = (un2f[:NP] + un2f[NP:])[:N]
    h2 = h2row[:N]
    un2 = un2[:N]
    den2t = den2[0, :N] + den2[1, :N] + wl2_p[:N]
    act2 = (un2 + wl2_p[:N] * h2) / (den2t + 1e-16) + b2
    act2 = jnp.where(act2 > 0, act2, jnp.expm1(act2))

    s = jax.ops.segment_sum(act2, batch, num_segments=G)
    cnt = jax.ops.segment_sum(jnp.ones((N, 1), jnp.float32), batch, num_segments=G)
    return s / jnp.maximum(cnt, 1.0)


# speedup vs baseline: 21.5022x; 1.6553x over previous
import functools

import jax
import jax.numpy as jnp
from jax import lax
from jax.experimental import pallas as pl
from jax.experimental.pallas import tpu as pltpu
from jax.experimental.pallas import tpu_sc as plsc

N = 50000
E = 800000
G = 1000
IN_DIM = 9
HIDDEN = 64
HEADS = 4

NP = 50176
EP = 819200
K = 3200
NTILES = 32
EB = EP // NTILES
NB = EB // K
CB = 16
R = 1568


def _edge_w_kernel(H):
    mesh = plsc.VectorSubcoreMesh(core_axis_name="c", subcore_axis_name="s")
    STRIPE = NP * H // 16
    CH = 3136
    NCH = STRIPE // CH

    def body(src_hbm, dst_hbm, as_hbm, ad_hbm,
             w_hbm, den_hbm,
             srcb, dstb, gidx, didx, asg, adg, wb, zb, den_sh):
        c = lax.axis_index("c")
        s = lax.axis_index("s")

        def zrow(g, _):
            zb[pl.ds(g * 16, 16)] = jnp.zeros((16,), jnp.float32)
            return 0
        lax.fori_loop(0, CH // 16, zrow, 0, unroll=8)
        for q in range(NCH):
            pltpu.sync_copy(zb, den_sh.at[pl.ds(s * STRIPE + q * CH, CH)])
        plsc.subcore_barrier()

        tile = c * 16 + s

        def batch_body(j, _):
            base = tile * EB + j * K
            pltpu.sync_copy(src_hbm.at[pl.ds(base, K)], srcb)
            pltpu.sync_copy(dst_hbm.at[pl.ds(base, K)], dstb)

            for h in range(H):
                if H > 1:
                    def off(g, _):
                        sl = pl.ds(g * 16, 16)
                        gidx[sl] = srcb[sl] + h * NP
                        didx[sl] = dstb[sl] + h * NP
                        return 0
                    lax.fori_loop(0, K // 16, off, 0, unroll=8)
                    gi, di = gidx, didx
                else:
                    gi, di = srcb, dstb
                pltpu.sync_copy(as_hbm.at[gi], asg)
                pltpu.sync_copy(ad_hbm.at[di], adg)

                def compute(g, _):
                    sl = pl.ds(g * 16, 16)
                    t = asg[sl] + adg[sl]
                    wb[sl] = jnp.exp(jnp.maximum(t, 0.2 * t))
                    return 0
                lax.fori_loop(0, K // 16, compute, 0, unroll=8)

                pltpu.sync_copy(wb, w_hbm.at[pl.ds(h * EP + base, K)])
                pltpu.sync_copy(wb, den_sh.at[di], add=True)
            return 0

        lax.fori_loop(0, NB, batch_body, 0)
        plsc.subcore_barrier()
        for q in range(NCH):
            pltpu.sync_copy(den_sh.at[pl.ds(s * STRIPE + q * CH, CH)], zb)
            pltpu.sync_copy(
                zb, den_hbm.at[pl.ds(c * (NP * H) + s * STRIPE + q * CH, CH)])

    scratch = [
        pltpu.VMEM((K,), jnp.int32),
        pltpu.VMEM((K,), jnp.int32),
        pltpu.VMEM((K,), jnp.int32),
        pltpu.VMEM((K,), jnp.int32),
        pltpu.VMEM((K,), jnp.float32),
        pltpu.VMEM((K,), jnp.float32),
        pltpu.VMEM((K,), jnp.float32),
        pltpu.VMEM((CH,), jnp.float32),
        pltpu.VMEM_SHARED((NP * H,), jnp.float32),
    ]
    return pl.kernel(
        body,
        mesh=mesh,
        out_type=[
            jax.ShapeDtypeStruct((EP * H,), jnp.float32),
            jax.ShapeDtypeStruct((2 * NP * H,), jnp.float32),
        ],
        scratch_types=scratch,
    )


def _msg_kernel(H):
    NCB = H * HIDDEN // CB
    HH = H * HIDDEN
    mesh = plsc.VectorSubcoreMesh(core_axis_name="c", subcore_axis_name="s")
    SROW = NP // 16
    WR = 784

    def body(src_hbm, dst_hbm, w_hbm, *rest):
        hcbs = rest[:NCB]
        un_hbm = rest[NCB]
        srcb, dstb, wbh, rows, zb, acc_sh = rest[NCB + 1:]
        c = lax.axis_index("c")
        s = lax.axis_index("s")
        tile = c * 16 + s

        def zrow(i, _):
            zb[i, pl.ds(0, 16)] = jnp.zeros((16,), jnp.float32)
            return 0
        lax.fori_loop(0, WR, zrow, 0, unroll=8)

        for cg in range(NCB):
            hd = cg * CB // HIDDEN
            for q in range(SROW // WR):
                pltpu.sync_copy(zb, acc_sh.at[pl.ds(s * SROW + q * WR, WR)])
            plsc.subcore_barrier()

            def batch_body(j, _):
                base = tile * EB + j * K
                pltpu.sync_copy(src_hbm.at[pl.ds(base, K)], srcb)
                pltpu.sync_copy(dst_hbm.at[pl.ds(base, K)], dstb)
                pltpu.sync_copy(w_hbm.at[pl.ds(hd * EP + base, K)],
                                wbh.at[pl.ds(0, K)])
                pltpu.sync_copy(hcbs[cg].at[srcb], rows)

                def mul(i, _):
                    w = wbh[pl.ds(i, 16)][0]
                    rows[i, pl.ds(0, 16)] = rows[i, pl.ds(0, 16)] * w
                    return 0
                lax.fori_loop(0, K, mul, 0, unroll=8)

                pltpu.sync_copy(rows, acc_sh.at[dstb], add=True)
                return 0

            lax.fori_loop(0, NB, batch_body, 0)
            plsc.subcore_barrier()

            for q in range(SROW // WR):
                row0 = s * SROW + q * WR
                pltpu.sync_copy(acc_sh.at[pl.ds(row0, WR)], rows.at[pl.ds(0, WR)])
                pltpu.sync_copy(rows.at[pl.ds(0, WR)],
                                un_hbm.at[pl.ds(c * NP + row0, WR),
                                          pl.ds(cg * CB, CB)])
            plsc.subcore_barrier()

    scratch = [
        pltpu.VMEM((K,), jnp.int32),
        pltpu.VMEM((K,), jnp.int32),
        pltpu.VMEM((K + 16,), jnp.float32),
        pltpu.VMEM((K, CB), jnp.float32),
        pltpu.VMEM((WR, CB), jnp.float32),
        pltpu.VMEM_SHARED((NP, CB), jnp.float32),
    ]
    return pl.kernel(
        body,
        mesh=mesh,
        out_type=jax.ShapeDtypeStruct((2 * NP, HH), jnp.float32),
        scratch_types=scratch,
        compiler_params=pltpu.CompilerParams(use_tc_tiling_on_sc=False),
    )


def _att_proj(att_src, att_dst):
    heads = att_src.shape[0]
    hh = heads * HIDDEN
    rows = jnp.arange(hh, dtype=jnp.int32)
    head_of_row = rows // HIDDEN
    A = jnp.zeros((hh, 2 * heads), dtype=jnp.float32)
    A = A.at[rows, head_of_row].set(att_src.reshape(-1))
    A = A.at[rows, heads + head_of_row].set(att_dst.reshape(-1))
    return A


NCB1 = HEADS * HIDDEN // CB
NCB2 = HIDDEN // CB


def _dense1_body(x_ref, w1_ref, a_ref, hrow_ref, *outs):
    hcb_refs = outs[:NCB1]
    asd_ref = outs[NCB1]
    wl_ref = outs[NCB1 + 1]
    h = jnp.dot(x_ref[...], w1_ref[...], preferred_element_type=jnp.float32)
    hrow_ref[...] = h
    for cg in range(NCB1):
        hcb_refs[cg][...] = h[:, cg * CB:(cg + 1) * CB]
    asd = jnp.dot(h, a_ref[...], preferred_element_type=jnp.float32)
    asd_ref[...] = asd
    t = asd[:, :HEADS] + asd[:, HEADS:]
    wl_ref[...] = jnp.exp(jnp.maximum(t, 0.2 * t))


def _dense1(x_p, W1, A1):
    grid = (NP // R,)
    hh = HEADS * HIDDEN
    blk = lambda cols: pl.BlockSpec((R, cols), lambda i: (i, 0))
    return pl.pallas_call(
        _dense1_body,
        grid=grid,
        in_specs=[
            blk(IN_DIM),
            pl.BlockSpec((IN_DIM, hh), lambda i: (0, 0)),
            pl.BlockSpec((hh, 2 * HEADS), lambda i: (0, 0)),
        ],
        out_specs=[blk(hh)] + [blk(CB)] * NCB1 + [blk(2 * HEADS), blk(HEADS)],
        out_shape=[jax.ShapeDtypeStruct((NP, hh), jnp.float32)]
        + [jax.ShapeDtypeStruct((NP, CB), jnp.float32)] * NCB1
        + [jax.ShapeDtypeStruct((NP, 2 * HEADS), jnp.float32),
           jax.ShapeDtypeStruct((NP, HEADS), jnp.float32)],
    )(x_p, W1, A1)


def _rep_mat(heads, out_ch):
    cols = jnp.arange(heads * out_ch, dtype=jnp.int32)
    return (cols[None, :] // out_ch == jnp.arange(heads, dtype=jnp.int32)[:, None]
            ).astype(jnp.float32)


def _dense2_body(una_ref, unb_ref, h1_ref, wl_ref, dena_ref, denb_ref,
                 rep_ref, b_ref, w2_ref, a_ref, *outs):
    h2row_ref = outs[0]
    h2cb_refs = outs[1:1 + NCB2]
    asd2_ref = outs[1 + NCB2]
    wl2_ref = outs[2 + NCB2]

    den = dena_ref[...] + denb_ref[...] + wl_ref[...]
    rden = 1.0 / (den + 1e-16)
    rden_rep = jnp.dot(rden, rep_ref[...], preferred_element_type=jnp.float32)
    wl_rep = jnp.dot(wl_ref[...], rep_ref[...], preferred_element_type=jnp.float32)

    act = (una_ref[...] + unb_ref[...]
           + wl_rep * h1_ref[...]) * rden_rep + b_ref[...]
    act = jnp.where(act > 0, act, jnp.exp(jnp.minimum(act, 0.0)) - 1.0)
    h2 = jnp.dot(act, w2_ref[...], preferred_element_type=jnp.float32)

    h2row_ref[...] = h2
    for cg in range(NCB2):
        h2cb_refs[cg][...] = h2[:, cg * CB:(cg + 1) * CB]
    asd = jnp.dot(h2, a_ref[...], preferred_element_type=jnp.float32)
    asd2_ref[...] = asd
    t = asd[:, :1] + asd[:, 1:]
    wl2_ref[...] = jnp.exp(jnp.maximum(t, 0.2 * t))


def _dense2(un1a, un1b, h1row, wl1, den1a, den1b, b1, W2, A2):
    grid = (NP // R,)
    hh = HEADS * HIDDEN
    rep = _rep_mat(HEADS, HIDDEN)
    blk = lambda cols: pl.BlockSpec((R, cols), lambda i: (i, 0))
    in_specs = [blk(hh), blk(hh), blk(hh),
                blk(HEADS), blk(HEADS), blk(HEADS),
                pl.BlockSpec((HEADS, hh), lambda i: (0, 0)),
                pl.BlockSpec((1, hh), lambda i: (0, 0)),
                pl.BlockSpec((hh, HIDDEN), lambda i: (0, 0)),
                pl.BlockSpec((HIDDEN, 2), lambda i: (0, 0))]
    return pl.pallas_call(
        _dense2_body,
        grid=grid,
        in_specs=in_specs,
        out_specs=[blk(HIDDEN)] + [blk(CB)] * NCB2 + [blk(2), blk(1)],
        out_shape=[jax.ShapeDtypeStruct((NP, HIDDEN), jnp.float32)]
        + [jax.ShapeDtypeStruct((NP, CB), jnp.float32)] * NCB2
        + [jax.ShapeDtypeStruct((NP, 2), jnp.float32),
           jax.ShapeDtypeStruct((NP, 1), jnp.float32)],
    )(un1a, un1b, h1row, wl1, den1a, den1b, rep, b1.reshape(1, hh), W2, A2)


def kernel(x, edge_index, batch, W1, att_src1, att_dst1, b1, W2, att_src2, att_dst2, b2):
    src = edge_index[0]
    dst = edge_index[1]
    pad_idx = N + jnp.arange(EP - E, dtype=jnp.int32) % (NP - N)
    pad_src = pad_idx
    pad_dst = pad_idx
    src_p = jnp.concatenate([src, pad_src])
    dst_p = jnp.concatenate([dst, pad_dst])
    x_p = jnp.zeros((NP, IN_DIM), jnp.float32).at[:N].set(x)

    A1 = _att_proj(att_src1, att_dst1)
    d1out = _dense1(x_p, W1, A1)
    h1row = d1out[0]
    hcb1 = d1out[1:1 + NCB1]
    asd1, wl1_p = d1out[1 + NCB1:]
    as1 = asd1[:, :HEADS]
    ad1 = asd1[:, HEADS:]

    w1f, den1f = _edge_w_kernel(HEADS)(
        src_p, dst_p, as1.T.reshape(-1), ad1.T.reshape(-1))
    den1 = den1f.reshape(2, HEADS, NP).transpose(0, 2, 1)

    un1f = _msg_kernel(HEADS)(src_p, dst_p, w1f, *hcb1)

    A2 = _att_proj(att_src2, att_dst2)
    d2out = _dense2(un1f[:NP], un1f[NP:], h1row, wl1_p, den1[0], den1[1],
                    b1, W2, A2)
    h2row = d2out[0]
    hcb2 = d2out[1:1 + NCB2]
    asd2, wl2_p = d2out[1 + NCB2:]
    as2 = asd2[:, :1]
    ad2 = asd2[:, 1:]

    w2f, den2f = _edge_w_kernel(1)(
        src_p, dst_p, as2.reshape(-1), ad2.reshape(-1))
    den2 = den2f.reshape(2, NP, 1)

    un2f = _msg_kernel(1)(src_p, dst_p, w2f, *hcb2)
    un2 = (un2f[:NP] + un2f[NP:])[:N]
    h2 = h2row[:N]
    un2 = un2[:N]
    den2t = den2[0, :N] + den2[1, :N] + wl2_p[:N]
    act2 = (un2 + wl2_p[:N] * h2) / (den2t + 1e-16) + b2
    act2 = jnp.where(act2 > 0, act2, jnp.expm1(act2))

    s = jax.ops.segment_sum(act2, batch, num_segments=G)
    cnt = jax.ops.segment_sum(jnp.ones((N, 1), jnp.float32), batch, num_segments=G)
    return s / jnp.maximum(cnt, 1.0)


# speedup vs baseline: 22.8584x; 1.0631x over previous
import functools

import jax
import jax.numpy as jnp
from jax import lax
from jax.experimental import pallas as pl
from jax.experimental.pallas import tpu as pltpu
from jax.experimental.pallas import tpu_sc as plsc

N = 50000
E = 800000
G = 1000
IN_DIM = 9
HIDDEN = 64
HEADS = 4

NP = 50176
EP = 819200
K = 1600
NTILES = 32
EB = EP // NTILES
NB = EB // K
CB = 16
R = 1568


def _edge_w_kernel(H):
    mesh = plsc.VectorSubcoreMesh(core_axis_name="c", subcore_axis_name="s")
    STRIPE = NP * H // 16
    CH = 3136
    NCH = STRIPE // CH

    def body(src_hbm, dst_hbm, as_hbm, ad_hbm,
             w_hbm, den_hbm,
             srcb, dstb, gidx, didx, asg, adg, wb, zb, den_sh):
        c = lax.axis_index("c")
        s = lax.axis_index("s")

        def zrow(g, _):
            zb[pl.ds(g * 16, 16)] = jnp.zeros((16,), jnp.float32)
            return 0
        lax.fori_loop(0, CH // 16, zrow, 0, unroll=8)
        for q in range(NCH):
            pltpu.sync_copy(zb, den_sh.at[pl.ds(s * STRIPE + q * CH, CH)])
        plsc.subcore_barrier()

        tile = c * 16 + s

        def batch_body(j, _):
            base = tile * EB + j * K
            pltpu.sync_copy(src_hbm.at[pl.ds(base, K)], srcb)
            pltpu.sync_copy(dst_hbm.at[pl.ds(base, K)], dstb)

            for h in range(H):
                if H > 1:
                    def off(g, _):
                        sl = pl.ds(g * 16, 16)
                        gidx[sl] = srcb[sl] + h * NP
                        didx[sl] = dstb[sl] + h * NP
                        return 0
                    lax.fori_loop(0, K // 16, off, 0, unroll=8)
                    gi, di = gidx, didx
                else:
                    gi, di = srcb, dstb
                pltpu.sync_copy(as_hbm.at[gi], asg)
                pltpu.sync_copy(ad_hbm.at[di], adg)

                def compute(g, _):
                    sl = pl.ds(g * 16, 16)
                    t = asg[sl] + adg[sl]
                    wb[sl] = jnp.exp(jnp.maximum(t, 0.2 * t))
                    return 0
                lax.fori_loop(0, K // 16, compute, 0, unroll=8)

                pltpu.sync_copy(wb, w_hbm.at[pl.ds(h * EP + base, K)])
                pltpu.sync_copy(wb, den_sh.at[di], add=True)
            return 0

        lax.fori_loop(0, NB, batch_body, 0)
        plsc.subcore_barrier()
        for q in range(NCH):
            pltpu.sync_copy(den_sh.at[pl.ds(s * STRIPE + q * CH, CH)], zb)
            pltpu.sync_copy(
                zb, den_hbm.at[pl.ds(c * (NP * H) + s * STRIPE + q * CH, CH)])

    scratch = [
        pltpu.VMEM((K,), jnp.int32),
        pltpu.VMEM((K,), jnp.int32),
        pltpu.VMEM((K,), jnp.int32),
        pltpu.VMEM((K,), jnp.int32),
        pltpu.VMEM((K,), jnp.float32),
        pltpu.VMEM((K,), jnp.float32),
        pltpu.VMEM((K,), jnp.float32),
        pltpu.VMEM((CH,), jnp.float32),
        pltpu.VMEM_SHARED((NP * H,), jnp.float32),
    ]
    return pl.kernel(
        body,
        mesh=mesh,
        out_type=[
            jax.ShapeDtypeStruct((EP * H,), jnp.float32),
            jax.ShapeDtypeStruct((2 * NP * H,), jnp.float32),
        ],
        scratch_types=scratch,
    )


def _msg_kernel(H):
    NCB = H * HIDDEN // CB
    HH = H * HIDDEN
    mesh = plsc.VectorSubcoreMesh(core_axis_name="c", subcore_axis_name="s")
    SROW = NP // 16
    WR = 392

    def body(src_hbm, dst_hbm, w_hbm, *rest):
        hcbs = rest[:NCB]
        un_hbm = rest[NCB]
        (srcb0, dstb0, wbh0, rows0,
         srcb1, dstb1, wbh1, rows1,
         zb, acc_sh, sem0, sem1) = rest[NCB + 1:]
        bufs = ((srcb0, dstb0, wbh0, rows0, sem0),
                (srcb1, dstb1, wbh1, rows1, sem1))
        c = lax.axis_index("c")
        s = lax.axis_index("s")
        tile = c * 16 + s

        def zrow(i, _):
            zb[i, pl.ds(0, 16)] = jnp.zeros((16,), jnp.float32)
            return 0
        lax.fori_loop(0, WR, zrow, 0, unroll=8)

        for cg in range(NCB):
            hd = cg * CB // HIDDEN

            def issue(j, b):
                sb, db, wb_, rw, sem = bufs[b]
                base = tile * EB + j * K
                pltpu.sync_copy(src_hbm.at[pl.ds(base, K)], sb)
                pltpu.sync_copy(dst_hbm.at[pl.ds(base, K)], db)
                pltpu.sync_copy(w_hbm.at[pl.ds(hd * EP + base, K)],
                                wb_.at[pl.ds(0, K)])
                pltpu.async_copy(hcbs[cg].at[sb], rw, sem)

            def finish(b):
                sb, db, wb_, rw, sem = bufs[b]
                pltpu.make_async_copy(hcbs[cg].at[pl.ds(0, K)], rw, sem).wait()

                def mul(i, _):
                    w = wb_[pl.ds(i, 16)][0]
                    rw[i, pl.ds(0, 16)] = rw[i, pl.ds(0, 16)] * w
                    return 0
                lax.fori_loop(0, K, mul, 0, unroll=8)
                pltpu.sync_copy(rw, acc_sh.at[db], add=True)

            for q in range(SROW // WR):
                pltpu.sync_copy(zb, acc_sh.at[pl.ds(s * SROW + q * WR, WR)])
            plsc.subcore_barrier()

            issue(0, 0)

            def pair(jj, _):
                j = jj * 2
                issue(j + 1, 1)
                finish(0)

                @pl.when(jj + 1 < NB // 2)
                def _():
                    issue(j + 2, 0)
                finish(1)
                return 0

            lax.fori_loop(0, NB // 2, pair, 0)
            plsc.subcore_barrier()

            for q in range(SROW // WR):
                row0 = s * SROW + q * WR
                pltpu.sync_copy(acc_sh.at[pl.ds(row0, WR)], rows0.at[pl.ds(0, WR)])
                pltpu.sync_copy(rows0.at[pl.ds(0, WR)],
                                un_hbm.at[pl.ds(c * NP + row0, WR),
                                          pl.ds(cg * CB, CB)])
            plsc.subcore_barrier()

    scratch = [
        pltpu.VMEM((K,), jnp.int32),
        pltpu.VMEM((K,), jnp.int32),
        pltpu.VMEM((K + 16,), jnp.float32),
        pltpu.VMEM((K, CB), jnp.float32),
        pltpu.VMEM((K,), jnp.int32),
        pltpu.VMEM((K,), jnp.int32),
        pltpu.VMEM((K + 16,), jnp.float32),
        pltpu.VMEM((K, CB), jnp.float32),
        pltpu.VMEM((WR, CB), jnp.float32),
        pltpu.VMEM_SHARED((NP, CB), jnp.float32),
        pltpu.SemaphoreType.DMA,
        pltpu.SemaphoreType.DMA,
    ]
    return pl.kernel(
        body,
        mesh=mesh,
        out_type=jax.ShapeDtypeStruct((2 * NP, HH), jnp.float32),
        scratch_types=scratch,
        compiler_params=pltpu.CompilerParams(use_tc_tiling_on_sc=False),
    )


def _att_proj(att_src, att_dst):
    heads = att_src.shape[0]
    hh = heads * HIDDEN
    rows = jnp.arange(hh, dtype=jnp.int32)
    head_of_row = rows // HIDDEN
    A = jnp.zeros((hh, 2 * heads), dtype=jnp.float32)
    A = A.at[rows, head_of_row].set(att_src.reshape(-1))
    A = A.at[rows, heads + head_of_row].set(att_dst.reshape(-1))
    return A


NCB1 = HEADS * HIDDEN // CB
NCB2 = HIDDEN // CB


def _dense1_body(x_ref, w1_ref, a_ref, hrow_ref, *outs):
    hcb_refs = outs[:NCB1]
    asd_ref = outs[NCB1]
    wl_ref = outs[NCB1 + 1]
    h = jnp.dot(x_ref[...], w1_ref[...], preferred_element_type=jnp.float32)
    hrow_ref[...] = h
    for cg in range(NCB1):
        hcb_refs[cg][...] = h[:, cg * CB:(cg + 1) * CB]
    asd = jnp.dot(h, a_ref[...], preferred_element_type=jnp.float32)
    asd_ref[...] = asd
    t = asd[:, :HEADS] + asd[:, HEADS:]
    wl_ref[...] = jnp.exp(jnp.maximum(t, 0.2 * t))


def _dense1(x_p, W1, A1):
    grid = (NP // R,)
    hh = HEADS * HIDDEN
    blk = lambda cols: pl.BlockSpec((R, cols), lambda i: (i, 0))
    return pl.pallas_call(
        _dense1_body,
        grid=grid,
        in_specs=[
            blk(IN_DIM),
            pl.BlockSpec((IN_DIM, hh), lambda i: (0, 0)),
            pl.BlockSpec((hh, 2 * HEADS), lambda i: (0, 0)),
        ],
        out_specs=[blk(hh)] + [blk(CB)] * NCB1 + [blk(2 * HEADS), blk(HEADS)],
        out_shape=[jax.ShapeDtypeStruct((NP, hh), jnp.float32)]
        + [jax.ShapeDtypeStruct((NP, CB), jnp.float32)] * NCB1
        + [jax.ShapeDtypeStruct((NP, 2 * HEADS), jnp.float32),
           jax.ShapeDtypeStruct((NP, HEADS), jnp.float32)],
    )(x_p, W1, A1)


def _rep_mat(heads, out_ch):
    cols = jnp.arange(heads * out_ch, dtype=jnp.int32)
    return (cols[None, :] // out_ch == jnp.arange(heads, dtype=jnp.int32)[:, None]
            ).astype(jnp.float32)


def _dense2_body(una_ref, unb_ref, h1_ref, wl_ref, dena_ref, denb_ref,
                 rep_ref, b_ref, w2_ref, a_ref, *outs):
    h2row_ref = outs[0]
    h2cb_refs = outs[1:1 + NCB2]
    asd2_ref = outs[1 + NCB2]
    wl2_ref = outs[2 + NCB2]

    den = dena_ref[...] + denb_ref[...] + wl_ref[...]
    rden = 1.0 / (den + 1e-16)
    rden_rep = jnp.dot(rden, rep_ref[...], preferred_element_type=jnp.float32)
    wl_rep = jnp.dot(wl_ref[...], rep_ref[...], preferred_element_type=jnp.float32)

    act = (una_ref[...] + unb_ref[...]
           + wl_rep * h1_ref[...]) * rden_rep + b_ref[...]
    act = jnp.where(act > 0, act, jnp.exp(jnp.minimum(act, 0.0)) - 1.0)
    h2 = jnp.dot(act, w2_ref[...], preferred_element_type=jnp.float32)

    h2row_ref[...] = h2
    for cg in range(NCB2):
        h2cb_refs[cg][...] = h2[:, cg * CB:(cg + 1) * CB]
    asd = jnp.dot(h2, a_ref[...], preferred_element_type=jnp.float32)
    asd2_ref[...] = asd
    t = asd[:, :1] + asd[:, 1:]
    wl2_ref[...] = jnp.exp(jnp.maximum(t, 0.2 * t))


def _dense2(un1a, un1b, h1row, wl1, den1a, den1b, b1, W2, A2):
    grid = (NP // R,)
    hh = HEADS * HIDDEN
    rep = _rep_mat(HEADS, HIDDEN)
    blk = lambda cols: pl.BlockSpec((R, cols), lambda i: (i, 0))
    in_specs = [blk(hh), blk(hh), blk(hh),
                blk(HEADS), blk(HEADS), blk(HEADS),
                pl.BlockSpec((HEADS, hh), lambda i: (0, 0)),
                pl.BlockSpec((1, hh), lambda i: (0, 0)),
                pl.BlockSpec((hh, HIDDEN), lambda i: (0, 0)),
                pl.BlockSpec((HIDDEN, 2), lambda i: (0, 0))]
    return pl.pallas_call(
        _dense2_body,
        grid=grid,
        in_specs=in_specs,
        out_specs=[blk(HIDDEN)] + [blk(CB)] * NCB2 + [blk(2), blk(1)],
        out_shape=[jax.ShapeDtypeStruct((NP, HIDDEN), jnp.float32)]
        + [jax.ShapeDtypeStruct((NP, CB), jnp.float32)] * NCB2
        + [jax.ShapeDtypeStruct((NP, 2), jnp.float32),
           jax.ShapeDtypeStruct((NP, 1), jnp.float32)],
    )(un1a, un1b, h1row, wl1, den1a, den1b, rep, b1.reshape(1, hh), W2, A2)


def kernel(x, edge_index, batch, W1, att_src1, att_dst1, b1, W2, att_src2, att_dst2, b2):
    src = edge_index[0]
    dst = edge_index[1]
    pad_idx = N + jnp.arange(EP - E, dtype=jnp.int32) % (NP - N)
    pad_src = pad_idx
    pad_dst = pad_idx
    src_p = jnp.concatenate([src, pad_src])
    dst_p = jnp.concatenate([dst, pad_dst])
    x_p = jnp.zeros((NP, IN_DIM), jnp.float32).at[:N].set(x)

    A1 = _att_proj(att_src1, att_dst1)
    d1out = _dense1(x_p, W1, A1)
    h1row = d1out[0]
    hcb1 = d1out[1:1 + NCB1]
    asd1, wl1_p = d1out[1 + NCB1:]
    as1 = asd1[:, :HEADS]
    ad1 = asd1[:, HEADS:]

    w1f, den1f = _edge_w_kernel(HEADS)(
        src_p, dst_p, as1.T.reshape(-1), ad1.T.reshape(-1))
    den1 = den1f.reshape(2, HEADS, NP).transpose(0, 2, 1)

    un1f = _msg_kernel(HEADS)(src_p, dst_p, w1f, *hcb1)

    A2 = _att_proj(att_src2, att_dst2)
    d2out = _dense2(un1f[:NP], un1f[NP:], h1row, wl1_p, den1[0], den1[1],
                    b1, W2, A2)
    h2row = d2out[0]
    hcb2 = d2out[1:1 + NCB2]
    asd2, wl2_p = d2out[1 + NCB2:]
    as2 = asd2[:, :1]
    ad2 = asd2[:, 1:]

    w2f, den2f = _edge_w_kernel(1)(
        src_p, dst_p, as2.reshape(-1), ad2.reshape(-1))
    den2 = den2f.reshape(2, NP, 1)

    un2f = _msg_kernel(1)(src_p, dst_p, w2f, *hcb2)
    un2 = (un2f[:NP] + un2f[NP:])[:N]
    h2 = h2row[:N]
    un2 = un2[:N]
    den2t = den2[0, :N] + den2[1, :N] + wl2_p[:N]
    act2 = (un2 + wl2_p[:N] * h2) / (den2t + 1e-16) + b2
    act2 = jnp.where(act2 > 0, act2, jnp.expm1(act2))

    s = jax.ops.segment_sum(act2, batch, num_segments=G)
    cnt = jax.ops.segment_sum(jnp.ones((N, 1), jnp.float32), batch, num_segments=G)
    return s / jnp.maximum(cnt, 1.0)
